# baseline (speedup 1.0000x reference)
import jax
import jax.numpy as jnp
from jax.experimental import pallas as pl
from jax.experimental.pallas import tpu as pltpu


def _round_up(x: int, m: int) -> int:
    return ((x + m - 1) // m) * m


def _gmf_tile_kernel(b_ref, w_ref, ue_ref, ie_ref, out_ref):
    h = ue_ref[...] * ie_ref[...] * w_ref[...]
    logit = jnp.sum(h, axis=1, keepdims=True)
    out_ref[...] = jax.nn.sigmoid(logit + b_ref[0])


def kernel(users, items, user_embedding, item_embedding, logit_w, logit_b):
    batch = users.shape[0]
    num_factors = user_embedding.shape[1]

    tb = 8192
    b_pad = _round_up(batch, tb)
    pad = b_pad - batch
    if pad:
        users = jnp.pad(users, (0, pad))
        items = jnp.pad(items, (0, pad))

    ue = jnp.take(user_embedding.astype(jnp.float32), users, axis=0)
    ie = jnp.take(item_embedding.astype(jnp.float32), items, axis=0)

    w_row = logit_w.reshape(1, -1).astype(jnp.float32)
    b = logit_b.reshape(-1).astype(jnp.float32)

    grid = (b_pad // tb,)
    out = pl.pallas_call(
        _gmf_tile_kernel,
        grid=grid,
        in_specs=[
            pl.BlockSpec(memory_space=pltpu.MemorySpace.SMEM),
            pl.BlockSpec((1, num_factors), lambda i: (0, 0)),
            pl.BlockSpec((tb, num_factors), lambda i: (i, 0)),
            pl.BlockSpec((tb, num_factors), lambda i: (i, 0)),
        ],
        out_specs=pl.BlockSpec((tb, 1), lambda i: (i, 0)),
        out_shape=jax.ShapeDtypeStruct((b_pad, 1), jnp.float32),
        compiler_params=pltpu.CompilerParams(
            dimension_semantics=("parallel",)),
    )(b, w_row, ue, ie)

    return out.reshape(-1)[:batch]
